# Initial kernel scaffold; baseline (speedup 1.0000x reference)
import jax
import jax.numpy as jnp
from jax.experimental import pallas as pl


def kernel(query, memory):
    raise NotImplementedError("write your pallas kernel here")


# speedup vs baseline: 1.9313x; 1.9313x over previous
import functools

import jax
import jax.numpy as jnp
from jax.experimental import pallas as pl
from jax.experimental.pallas import tpu as pltpu


def _flash_body(num_tiles, q_ref, m_ref, o_ref, acc_ref, mx_ref, l_ref):
    i = pl.program_id(0)

    @pl.when(i == 0)
    def _init():
        acc_ref[...] = jnp.zeros_like(acc_ref)
        mx_ref[...] = jnp.full_like(mx_ref, -1e30)
        l_ref[...] = jnp.zeros_like(l_ref)

    q = q_ref[...]
    m = m_ref[...]
    s = jax.lax.dot_general(
        q, m, (((1,), (1,)), ((), ())), preferred_element_type=jnp.float32
    )

    m_prev = mx_ref[...]
    m_cur = jnp.max(s, axis=1, keepdims=True)
    m_new = jnp.maximum(m_prev, m_cur)
    corr = jnp.exp(m_prev - m_new)
    p = jnp.exp(s - m_new[:, 0:1])
    l_cur = jnp.sum(p, axis=1, keepdims=True)

    l_ref[...] = l_ref[...] * corr + l_cur
    mx_ref[...] = m_new
    acc_ref[...] = acc_ref[...] * corr + jnp.dot(
        p, m, preferred_element_type=jnp.float32
    )

    @pl.when(i == num_tiles - 1)
    def _finish():
        o_ref[...] = acc_ref[...] / l_ref[...]


@functools.partial(jax.jit, static_argnames=("tile",))
def _content_addressed_read(query, memory, tile=4096):
    batch, dim = query.shape
    num_slots = memory.shape[0]
    num_tiles = num_slots // tile

    return pl.pallas_call(
        functools.partial(_flash_body, num_tiles),
        grid=(num_tiles,),
        in_specs=[
            pl.BlockSpec((batch, dim), lambda i: (0, 0)),
            pl.BlockSpec((tile, dim), lambda i: (i, 0)),
        ],
        out_specs=pl.BlockSpec((batch, dim), lambda i: (0, 0)),
        out_shape=jax.ShapeDtypeStruct((batch, dim), jnp.float32),
        scratch_shapes=[
            pltpu.VMEM((batch, dim), jnp.float32),
            pltpu.VMEM((batch, dim), jnp.float32),
            pltpu.VMEM((batch, dim), jnp.float32),
        ],
    )(query, memory)


def kernel(query, memory):
    return _content_addressed_read(query, memory)


# speedup vs baseline: 2.4175x; 1.2518x over previous
import functools

import jax
import jax.numpy as jnp
from jax.experimental import pallas as pl
from jax.experimental.pallas import tpu as pltpu


def _flash_body(num_tiles, q_ref, m_ref, o_ref, acc_ref, mx_ref, l_ref):
    i = pl.program_id(0)

    @pl.when(i == 0)
    def _init():
        acc_ref[...] = jnp.zeros_like(acc_ref)
        mx_ref[...] = jnp.full_like(mx_ref, -1e30)
        l_ref[...] = jnp.zeros_like(l_ref)

    q = q_ref[...]
    m = m_ref[...]
    s = jax.lax.dot_general(
        q, m, (((1,), (1,)), ((), ())), preferred_element_type=jnp.float32
    )

    m_prev = mx_ref[...]
    m_cur = jnp.max(s, axis=1, keepdims=True)
    m_new = jnp.maximum(m_prev, m_cur)
    corr = jnp.exp(m_prev - m_new)
    p = jnp.exp(s - m_new[:, 0:1])
    l_cur = jnp.sum(p, axis=1, keepdims=True)

    l_ref[...] = l_ref[...] * corr + l_cur
    mx_ref[...] = m_new
    acc_ref[...] = acc_ref[...] * corr + jnp.dot(
        p, m, preferred_element_type=jnp.float32
    )

    @pl.when(i == num_tiles - 1)
    def _finish():
        o_ref[...] = acc_ref[...] / l_ref[...]


@functools.partial(jax.jit, static_argnames=("tile",))
def _content_addressed_read(query, memory, tile=8192):
    batch, dim = query.shape
    num_slots = memory.shape[0]
    num_tiles = num_slots // tile

    return pl.pallas_call(
        functools.partial(_flash_body, num_tiles),
        grid=(num_tiles,),
        in_specs=[
            pl.BlockSpec((batch, dim), lambda i: (0, 0)),
            pl.BlockSpec((tile, dim), lambda i: (i, 0)),
        ],
        out_specs=pl.BlockSpec((batch, dim), lambda i: (0, 0)),
        out_shape=jax.ShapeDtypeStruct((batch, dim), jnp.float32),
        scratch_shapes=[
            pltpu.VMEM((batch, dim), jnp.float32),
            pltpu.VMEM((batch, dim), jnp.float32),
            pltpu.VMEM((batch, dim), jnp.float32),
        ],
    )(query, memory)


def kernel(query, memory):
    return _content_addressed_read(query, memory)


# speedup vs baseline: 2.6829x; 1.1098x over previous
import functools

import jax
import jax.numpy as jnp
from jax.experimental import pallas as pl
from jax.experimental.pallas import tpu as pltpu


def _flash_body(num_tiles, q_ref, m_ref, o_ref, acc_ref, mx_ref, l_ref):
    i = pl.program_id(0)

    @pl.when(i == 0)
    def _init():
        acc_ref[...] = jnp.zeros_like(acc_ref)
        mx_ref[...] = jnp.full_like(mx_ref, -1e30)
        l_ref[...] = jnp.zeros_like(l_ref)

    q = q_ref[...]
    m = m_ref[...]
    s = jax.lax.dot_general(
        q, m, (((1,), (1,)), ((), ())), preferred_element_type=jnp.float32
    )

    m_prev = mx_ref[...]
    m_cur = jnp.max(s, axis=1, keepdims=True)
    m_new = jnp.maximum(m_prev, m_cur)
    corr = jnp.exp(m_prev - m_new)
    p = jnp.exp(s - m_new[:, 0:1])
    l_cur = jnp.sum(p, axis=1, keepdims=True)

    l_ref[...] = l_ref[...] * corr + l_cur
    mx_ref[...] = m_new
    acc_ref[...] = acc_ref[...] * corr + jnp.dot(
        p, m, preferred_element_type=jnp.float32
    )

    @pl.when(i == num_tiles - 1)
    def _finish():
        o_ref[...] = acc_ref[...] / l_ref[...]


@functools.partial(jax.jit, static_argnames=("tile",))
def _content_addressed_read(query, memory, tile=16384):
    batch, dim = query.shape
    num_slots = memory.shape[0]
    num_tiles = num_slots // tile

    return pl.pallas_call(
        functools.partial(_flash_body, num_tiles),
        grid=(num_tiles,),
        in_specs=[
            pl.BlockSpec((batch, dim), lambda i: (0, 0)),
            pl.BlockSpec((tile, dim), lambda i: (i, 0)),
        ],
        out_specs=pl.BlockSpec((batch, dim), lambda i: (0, 0)),
        out_shape=jax.ShapeDtypeStruct((batch, dim), jnp.float32),
        scratch_shapes=[
            pltpu.VMEM((batch, dim), jnp.float32),
            pltpu.VMEM((batch, dim), jnp.float32),
            pltpu.VMEM((batch, dim), jnp.float32),
        ],
    )(query, memory)


def kernel(query, memory):
    return _content_addressed_read(query, memory)
